# baseline (speedup 1.0000x reference)
import functools

import jax
import jax.numpy as jnp
from jax import lax
from jax.experimental import pallas as pl
from jax.experimental.pallas import tpu as pltpu

_HEADS = 4
_DIM_HEAD = 32
_DIM = 128
_TOPK_RATIO = 0.25
_MIN_TOPK = 32
_QUERY_KEEP_RATIO = 0.75
_MIN_QUERY_KEEP = 64


def _f32_sort_keys(x):
    b = lax.bitcast_convert_type(x + 0.0, jnp.int32)
    b = b ^ ((b >> 31) & jnp.int32(0x7FFFFFFF))
    return lax.bitcast_convert_type(b, jnp.uint32) ^ jnp.uint32(0x80000000)


def _topk_mask(x, kk, axis):
    assert axis == 0
    u = _f32_sort_keys(x)
    C = x.shape[axis]
    red_shape = tuple(1 if a == axis else s for a, s in enumerate(x.shape))
    kf = jnp.float32(kk)
    ones_row = jnp.ones((1, x.shape[0]), jnp.float32)

    def count_gt(t):
        return _mm(ones_row, (u > t).astype(jnp.float32))

    lo0 = jnp.zeros(red_shape, jnp.uint32)
    hi0 = jnp.full(red_shape, jnp.uint32(0xFFFFFFFF))

    def vbody(_, carry):
        lo, hi = carry
        mid = lo + ((hi - lo) >> jnp.uint32(1))
        small = count_gt(mid) < kf
        return (jnp.where(small, lo, mid + jnp.uint32(1)),
                jnp.where(small, mid, hi))

    lo, _ = lax.fori_loop(0, 32, vbody, (lo0, hi0))
    thr = lo
    gt = u > thr
    eq = u == thr
    eqf = eq.astype(jnp.float32)
    r = kf - jnp.sum(gt.astype(jnp.float32), axis=axis, keepdims=True)
    idx = lax.broadcasted_iota(jnp.int32, x.shape, axis)

    ilo0 = jnp.zeros(red_shape, jnp.int32)
    ihi0 = jnp.full(red_shape, jnp.int32(C))

    def ibody(_, carry):
        ilo, ihi = carry
        mid = ilo + ((ihi - ilo) >> 1)
        g = _mm(ones_row, eqf * (idx < mid).astype(jnp.float32))
        ge = g >= r
        return (jnp.where(ge, ilo, mid + 1), jnp.where(ge, mid, ihi))

    iters = max(1, C.bit_length())
    ilo, _ = lax.fori_loop(0, iters, ibody, (ilo0, ihi0))
    return gt | (eq & (idx < ilo))


def _key_to_bf16(kint):
    s = kint ^ jnp.int32(0x8000)
    flip = jnp.where(s >= 32768, jnp.int32(0x7FFF), jnp.int32(0))
    patt = (s ^ flip) << 16
    return lax.bitcast_convert_type(patt, jnp.float32).astype(jnp.bfloat16)


def _kth_largest_bf16(xb, kk):
    R, C = xb.shape
    kf = jnp.float32(kk)
    one = jnp.bfloat16(1.0)
    zero = jnp.bfloat16(0.0)
    ones_bf = jnp.ones((C, 1), jnp.bfloat16)
    def body(_, carry):
        lo, hi = carry
        mid = lo + ((hi - lo) >> 1)
        tb = _key_to_bf16(mid)
        cf = _mm(jnp.where(xb > tb, one, zero), ones_bf)
        small = cf < kf
        return (jnp.where(small, lo, mid + 1), jnp.where(small, mid, hi))

    carry = (jnp.zeros((R, 1), jnp.int32), jnp.full((R, 1), jnp.int32(65535)))
    for i in range(16):
        carry = body(i, carry)
    return _key_to_bf16(carry[0])


def _mm(a, b):
    return lax.dot_general(a, b, (((1,), (0,)), ((), ())),
                           preferred_element_type=jnp.float32)


def _mm_t(a, b):
    return lax.dot_general(a, b, (((1,), (1,)), ((), ())),
                           preferred_element_type=jnp.float32)


def _body(q_ref, k_ref, v_ref, skip_ref, lnqg, lnqb, lnkg, lnkb, lnvg, lnvb,
          Wq_ref, bq_ref, Wk_ref, bk_ref, Wv_ref, bv_ref, Wp_ref, bp_ref,
          out_ref, *, n_cam, keep_q, keep_k):
    Tq = q_ref.shape[1] * q_ref.shape[4] * q_ref.shape[5]
    Tk = k_ref.shape[1] * k_ref.shape[4] * k_ref.shape[5]
    qx = q_ref[...].reshape(Tq, _DIM)
    kx = k_ref[...].reshape(Tk, _DIM)
    vx = v_ref[...].reshape(Tk, _DIM)

    def ln(x, g, b):
        m = jnp.mean(x, axis=-1, keepdims=True)
        v = jnp.mean((x - m) ** 2, axis=-1, keepdims=True)
        return (x - m) / jnp.sqrt(v + 1e-5) * g[...] + b[...]

    qf = _mm(ln(qx, lnqg, lnqb), Wq_ref[...]) + bq_ref[...]
    kf = _mm(ln(kx, lnkg, lnkb), Wk_ref[...]) + bk_ref[...]
    vf = _mm(ln(vx, lnvg, lnvb), Wv_ref[...]) + bv_ref[...]

    d_iota = lax.broadcasted_iota(jnp.int32, (_DIM, _HEADS), 0)
    h_iota = lax.broadcasted_iota(jnp.int32, (_DIM, _HEADS), 1)
    sel = (d_iota // _DIM_HEAD == h_iota).astype(jnp.float32)
    sal = _mm(qf * qf, sel)
    qmask = _topk_mask(sal, keep_q, axis=0)
    qmf = qmask.astype(jnp.float32) * jnp.float32(_DIM_HEAD ** -0.5)

    Tk = kf.shape[0]
    ones_bf = jnp.ones((Tk, 1), jnp.bfloat16)
    kidx = lax.broadcasted_iota(jnp.int32, (Tq, Tk), 1)
    negiota = (-kidx).astype(jnp.bfloat16)
    vfb = vf.astype(jnp.bfloat16)

    kfb = kf.astype(jnp.bfloat16)
    logits_l, xb_l = [], []
    for h in range(_HEADS):
        sl = slice(h * _DIM_HEAD, (h + 1) * _DIM_HEAD)
        qhb = (qf[:, sl] * qmf[:, h:h + 1]).astype(jnp.bfloat16)
        logits = _mm_t(qhb, kfb[:, sl])
        logits_l.append(logits)
        xb_l.append(jnp.where(qmask[:, h:h + 1], logits.astype(jnp.bfloat16),
                              negiota))

    thr_all = _kth_largest_bf16(jnp.concatenate(xb_l, axis=0), keep_k)

    heads = []
    for h in range(_HEADS):
        sl = slice(h * _DIM_HEAD, (h + 1) * _DIM_HEAD)
        xb, logits = xb_l[h], logits_l[h]
        thr = thr_all[h * Tq:(h + 1) * Tq]
        shift = jnp.maximum(thr.astype(jnp.float32), 0.0)
        exb = jnp.where(xb >= thr, jnp.exp(logits - shift),
                        0.0).astype(jnp.bfloat16)
        sums = _mm(exb, ones_bf)
        o = _mm(exb, vfb[:, sl])
        heads.append(o / sums)

    z = _mm(jnp.concatenate(heads, axis=1), Wp_ref[...]) + bp_ref[...]
    z = jnp.mean(z.reshape(n_cam, Tq // n_cam, _DIM), axis=0)
    z = z + skip_ref[...].reshape(Tq // n_cam, _DIM)
    out_ref[...] = z.reshape(out_ref.shape)


def kernel(q, k, v, skip, ln_q_g, ln_q_b, ln_k_g, ln_k_b, ln_v_g, ln_v_b,
           Wq, bq, Wk, bk, Wv, bv, Wp, bp):
    b, n, qH, qW, qw1, qw2, d = q.shape
    _, _, kH, kW, kw1, kw2, _ = k.shape
    Tq = n * qw1 * qw2
    Tk = n * kw1 * kw2
    keep_q = min(max(max(int(Tq * _QUERY_KEEP_RATIO), _MIN_QUERY_KEEP), 1), Tq)
    keep_k = min(max(int(Tk * _TOPK_RATIO), _MIN_TOPK), Tk)

    row = lambda a: a.reshape(1, -1)
    params = dict(
        lnqg=row(ln_q_g), lnqb=row(ln_q_b), lnkg=row(ln_k_g),
        lnkb=row(ln_k_b), lnvg=row(ln_v_g), lnvb=row(ln_v_b),
        bq=row(bq), bk=row(bk), bv=row(bv), bp=row(bp))

    def pspec(shape):
        return pl.BlockSpec(shape, lambda ib, ix, iy: (0,) * len(shape))

    grid = (b, qH, qW)
    out = pl.pallas_call(
        functools.partial(_body, n_cam=n, keep_q=keep_q, keep_k=keep_k),
        grid=grid,
        in_specs=[
            pl.BlockSpec((1, n, 1, 1, qw1, qw2, d),
                         lambda ib, ix, iy: (ib, 0, ix, iy, 0, 0, 0)),
            pl.BlockSpec((1, n, 1, 1, kw1, kw2, d),
                         lambda ib, ix, iy: (ib, 0, ix, iy, 0, 0, 0)),
            pl.BlockSpec((1, n, 1, 1, kw1, kw2, d),
                         lambda ib, ix, iy: (ib, 0, ix, iy, 0, 0, 0)),
            pl.BlockSpec((1, 1, 1, qw1, qw2, d),
                         lambda ib, ix, iy: (ib, ix, iy, 0, 0, 0)),
            pspec((1, d)), pspec((1, d)), pspec((1, d)), pspec((1, d)),
            pspec((1, d)), pspec((1, d)),
            pspec((d, _HEADS * _DIM_HEAD)), pspec((1, _HEADS * _DIM_HEAD)),
            pspec((d, _HEADS * _DIM_HEAD)), pspec((1, _HEADS * _DIM_HEAD)),
            pspec((d, _HEADS * _DIM_HEAD)), pspec((1, _HEADS * _DIM_HEAD)),
            pspec((_HEADS * _DIM_HEAD, d)), pspec((1, d)),
        ],
        out_specs=pl.BlockSpec((1, 1, 1, qw1, qw2, d),
                               lambda ib, ix, iy: (ib, ix, iy, 0, 0, 0)),
        out_shape=jax.ShapeDtypeStruct((b, qH, qW, qw1, qw2, d), jnp.float32),
        compiler_params=pltpu.CompilerParams(
            dimension_semantics=("parallel", "parallel", "parallel")),
    )(q, k, v, skip, params["lnqg"], params["lnqb"], params["lnkg"],
      params["lnkb"], params["lnvg"], params["lnvb"], Wq, params["bq"],
      Wk, params["bk"], Wv, params["bv"], Wp, params["bp"])
    return out


# speedup vs baseline: 1.0951x; 1.0951x over previous
import functools

import jax
import jax.numpy as jnp
from jax import lax
from jax.experimental import pallas as pl
from jax.experimental.pallas import tpu as pltpu

_HEADS = 4
_DIM_HEAD = 32
_DIM = 128
_TOPK_RATIO = 0.25
_MIN_TOPK = 32
_QUERY_KEEP_RATIO = 0.75
_MIN_QUERY_KEEP = 64


def _f32_sort_keys(x):
    b = lax.bitcast_convert_type(x + 0.0, jnp.int32)
    b = b ^ ((b >> 31) & jnp.int32(0x7FFFFFFF))
    return lax.bitcast_convert_type(b, jnp.uint32) ^ jnp.uint32(0x80000000)


def _topk_mask(x, kk, axis):
    assert axis == 0
    u = _f32_sort_keys(x)
    C = x.shape[axis]
    red_shape = tuple(1 if a == axis else s for a, s in enumerate(x.shape))
    kf = jnp.float32(kk)
    ones_row = jnp.ones((1, x.shape[0]), jnp.float32)

    def count_gt(t):
        return _mm(ones_row, (u > t).astype(jnp.float32))

    lo0 = jnp.zeros(red_shape, jnp.uint32)
    hi0 = jnp.full(red_shape, jnp.uint32(0xFFFFFFFF))

    def vbody(_, carry):
        lo, hi = carry
        mid = lo + ((hi - lo) >> jnp.uint32(1))
        small = count_gt(mid) < kf
        return (jnp.where(small, lo, mid + jnp.uint32(1)),
                jnp.where(small, mid, hi))

    lo, _ = lax.fori_loop(0, 32, vbody, (lo0, hi0))
    thr = lo
    gt = u > thr
    eq = u == thr
    eqf = eq.astype(jnp.float32)
    r = kf - jnp.sum(gt.astype(jnp.float32), axis=axis, keepdims=True)
    idx = lax.broadcasted_iota(jnp.int32, x.shape, axis)

    ilo0 = jnp.zeros(red_shape, jnp.int32)
    ihi0 = jnp.full(red_shape, jnp.int32(C))

    def ibody(_, carry):
        ilo, ihi = carry
        mid = ilo + ((ihi - ilo) >> 1)
        g = _mm(ones_row, eqf * (idx < mid).astype(jnp.float32))
        ge = g >= r
        return (jnp.where(ge, ilo, mid + 1), jnp.where(ge, mid, ihi))

    iters = max(1, C.bit_length())
    ilo, _ = lax.fori_loop(0, iters, ibody, (ilo0, ihi0))
    return gt | (eq & (idx < ilo))


def _key_to_bf16(kint):
    s = kint ^ jnp.int32(0x8000)
    flip = jnp.where(s >= 32768, jnp.int32(0x7FFF), jnp.int32(0))
    patt = (s ^ flip) << 16
    return lax.bitcast_convert_type(patt, jnp.float32).astype(jnp.bfloat16)


def _kth_largest_bf16(xb, kk):
    R, C = xb.shape
    kf = jnp.float32(kk)
    one = jnp.bfloat16(1.0)
    zero = jnp.bfloat16(0.0)
    ones_bf = jnp.ones((C, 1), jnp.bfloat16)
    def body(_, carry):
        lo, hi = carry
        mid = lo + ((hi - lo) >> 1)
        tb = _key_to_bf16(mid)
        cf = _mm(jnp.where(xb > tb, one, zero), ones_bf)
        small = cf < kf
        return (jnp.where(small, lo, mid + 1), jnp.where(small, mid, hi))

    lo, _ = lax.fori_loop(
        0, 16, body,
        (jnp.zeros((R, 1), jnp.int32), jnp.full((R, 1), jnp.int32(65535))))
    return _key_to_bf16(lo)


def _mm(a, b):
    return lax.dot_general(a, b, (((1,), (0,)), ((), ())),
                           preferred_element_type=jnp.float32)


def _mm_t(a, b):
    return lax.dot_general(a, b, (((1,), (1,)), ((), ())),
                           preferred_element_type=jnp.float32)


def _body(q_ref, k_ref, v_ref, skip_ref, lnqg, lnqb, lnkg, lnkb, lnvg, lnvb,
          Wq_ref, bq_ref, Wk_ref, bk_ref, Wv_ref, bv_ref, Wp_ref, bp_ref,
          out_ref, *, n_cam, keep_q, keep_k):
    Tq = q_ref.shape[1] * q_ref.shape[4] * q_ref.shape[5]
    Tk = k_ref.shape[1] * k_ref.shape[4] * k_ref.shape[5]
    qx = q_ref[...].reshape(Tq, _DIM)
    kx = k_ref[...].reshape(Tk, _DIM)
    vx = v_ref[...].reshape(Tk, _DIM)

    def ln(x, g, b):
        m = jnp.mean(x, axis=-1, keepdims=True)
        v = jnp.mean((x - m) ** 2, axis=-1, keepdims=True)
        return (x - m) / jnp.sqrt(v + 1e-5) * g[...] + b[...]

    qf = _mm(ln(qx, lnqg, lnqb), Wq_ref[...]) + bq_ref[...]
    kf = _mm(ln(kx, lnkg, lnkb), Wk_ref[...]) + bk_ref[...]
    vf = _mm(ln(vx, lnvg, lnvb), Wv_ref[...]) + bv_ref[...]

    d_iota = lax.broadcasted_iota(jnp.int32, (_DIM, _HEADS), 0)
    h_iota = lax.broadcasted_iota(jnp.int32, (_DIM, _HEADS), 1)
    sel = (d_iota // _DIM_HEAD == h_iota).astype(jnp.float32)
    sal = _mm(qf * qf, sel)
    qmask = _topk_mask(sal, keep_q, axis=0)
    qmf = qmask.astype(jnp.float32) * jnp.float32(_DIM_HEAD ** -0.5)

    Tk = kf.shape[0]
    ones_bf = jnp.ones((Tk, 1), jnp.bfloat16)
    kidx = lax.broadcasted_iota(jnp.int32, (Tq, Tk), 1)
    negiota = (-kidx).astype(jnp.bfloat16)
    vfb = vf.astype(jnp.bfloat16)

    kfb = kf.astype(jnp.bfloat16)
    logits_l, xb_l = [], []
    for h in range(_HEADS):
        sl = slice(h * _DIM_HEAD, (h + 1) * _DIM_HEAD)
        qhb = (qf[:, sl] * qmf[:, h:h + 1]).astype(jnp.bfloat16)
        logits = _mm_t(qhb, kfb[:, sl])
        logits_l.append(logits)
        xb_l.append(jnp.where(qmask[:, h:h + 1], logits.astype(jnp.bfloat16),
                              negiota))

    thr_all = _kth_largest_bf16(jnp.concatenate(xb_l, axis=0), keep_k)

    heads = []
    for h in range(_HEADS):
        sl = slice(h * _DIM_HEAD, (h + 1) * _DIM_HEAD)
        xb, logits = xb_l[h], logits_l[h]
        thr = thr_all[h * Tq:(h + 1) * Tq]
        shift = jnp.maximum(thr.astype(jnp.float32), 0.0)
        exb = jnp.where(xb >= thr, jnp.exp(logits - shift),
                        0.0).astype(jnp.bfloat16)
        sums = _mm(exb, ones_bf)
        o = _mm(exb, vfb[:, sl])
        heads.append(o / sums)

    z = _mm(jnp.concatenate(heads, axis=1), Wp_ref[...]) + bp_ref[...]
    z = jnp.mean(z.reshape(n_cam, Tq // n_cam, _DIM), axis=0)
    z = z + skip_ref[...].reshape(Tq // n_cam, _DIM)
    out_ref[...] = z.reshape(out_ref.shape)


def kernel(q, k, v, skip, ln_q_g, ln_q_b, ln_k_g, ln_k_b, ln_v_g, ln_v_b,
           Wq, bq, Wk, bk, Wv, bv, Wp, bp):
    b, n, qH, qW, qw1, qw2, d = q.shape
    _, _, kH, kW, kw1, kw2, _ = k.shape
    Tq = n * qw1 * qw2
    Tk = n * kw1 * kw2
    keep_q = min(max(max(int(Tq * _QUERY_KEEP_RATIO), _MIN_QUERY_KEEP), 1), Tq)
    keep_k = min(max(int(Tk * _TOPK_RATIO), _MIN_TOPK), Tk)

    row = lambda a: a.reshape(1, -1)
    params = dict(
        lnqg=row(ln_q_g), lnqb=row(ln_q_b), lnkg=row(ln_k_g),
        lnkb=row(ln_k_b), lnvg=row(ln_v_g), lnvb=row(ln_v_b),
        bq=row(bq), bk=row(bk), bv=row(bv), bp=row(bp))

    def pspec(shape):
        return pl.BlockSpec(shape, lambda ib, ix, iy: (0,) * len(shape))

    grid = (b, qH, qW)
    out = pl.pallas_call(
        functools.partial(_body, n_cam=n, keep_q=keep_q, keep_k=keep_k),
        grid=grid,
        in_specs=[
            pl.BlockSpec((1, n, 1, 1, qw1, qw2, d),
                         lambda ib, ix, iy: (ib, 0, ix, iy, 0, 0, 0)),
            pl.BlockSpec((1, n, 1, 1, kw1, kw2, d),
                         lambda ib, ix, iy: (ib, 0, ix, iy, 0, 0, 0)),
            pl.BlockSpec((1, n, 1, 1, kw1, kw2, d),
                         lambda ib, ix, iy: (ib, 0, ix, iy, 0, 0, 0)),
            pl.BlockSpec((1, 1, 1, qw1, qw2, d),
                         lambda ib, ix, iy: (ib, ix, iy, 0, 0, 0)),
            pspec((1, d)), pspec((1, d)), pspec((1, d)), pspec((1, d)),
            pspec((1, d)), pspec((1, d)),
            pspec((d, _HEADS * _DIM_HEAD)), pspec((1, _HEADS * _DIM_HEAD)),
            pspec((d, _HEADS * _DIM_HEAD)), pspec((1, _HEADS * _DIM_HEAD)),
            pspec((d, _HEADS * _DIM_HEAD)), pspec((1, _HEADS * _DIM_HEAD)),
            pspec((_HEADS * _DIM_HEAD, d)), pspec((1, d)),
        ],
        out_specs=pl.BlockSpec((1, 1, 1, qw1, qw2, d),
                               lambda ib, ix, iy: (ib, ix, iy, 0, 0, 0)),
        out_shape=jax.ShapeDtypeStruct((b, qH, qW, qw1, qw2, d), jnp.float32),
        compiler_params=pltpu.CompilerParams(
            dimension_semantics=("parallel", "parallel", "parallel")),
    )(q, k, v, skip, params["lnqg"], params["lnqb"], params["lnkg"],
      params["lnkb"], params["lnvg"], params["lnvb"], Wq, params["bq"],
      Wk, params["bk"], Wv, params["bv"], Wp, params["bp"])
    return out
